# Initial kernel scaffold; baseline (speedup 1.0000x reference)
import jax
import jax.numpy as jnp
from jax.experimental import pallas as pl


def kernel(x, edge_index, W, b):
    raise NotImplementedError("write your pallas kernel here")


# speedup vs baseline: 15.5662x; 15.5662x over previous
import functools

import jax
import jax.numpy as jnp
from jax import lax
from jax.experimental import pallas as pl
from jax.experimental.pallas import tpu as pltpu
from jax.experimental.pallas import tpu_sc as plsc

NC = 2
NS = 16
NW = NC * NS
CHUNK = 128


def _sc_mesh():
    return plsc.VectorSubcoreMesh(core_axis_name="c", subcore_axis_name="s")


def _deg_body(k, stripe, dst_hbm, ones_hbm, zeros_hbm, out_hbm,
              dst_v, ones_v, bounce_v, deg_sp):
    c = lax.axis_index("c")
    s = lax.axis_index("s")
    wid = s * NC + c
    pltpu.sync_copy(dst_hbm.at[wid], dst_v)
    pltpu.sync_copy(ones_hbm, ones_v)
    pltpu.sync_copy(zeros_hbm, bounce_v)
    for t in range(stripe // CHUNK):
        pltpu.sync_copy(bounce_v, deg_sp.at[pl.ds(s * stripe + t * CHUNK, CHUNK)])
    plsc.subcore_barrier()

    def body(j, carry):
        pltpu.sync_copy(ones_v, deg_sp.at[dst_v.at[j]], add=True)
        return carry

    lax.fori_loop(0, k, body, 0)
    plsc.subcore_barrier()
    for t in range(stripe // CHUNK):
        base = s * stripe + t * CHUNK
        pltpu.sync_copy(deg_sp.at[pl.ds(base, CHUNK)], bounce_v)
        pltpu.sync_copy(bounce_v, out_hbm.at[c, pl.ds(base, CHUNK)])


def _edge_body(k, stripe, hs_hbm, src_hbm, dst_hbm, zeros_hbm,
               out_hbm, src_v, dst_v, buf_a, acc_sp, sem_a):
    c = lax.axis_index("c")
    s = lax.axis_index("s")
    wid = s * NC + c
    pltpu.sync_copy(src_hbm.at[wid], src_v)
    pltpu.sync_copy(dst_hbm.at[wid], dst_v)
    pltpu.sync_copy(zeros_hbm, buf_a)
    for t in range(stripe // CHUNK):
        pltpu.sync_copy(buf_a, acc_sp.at[pl.ds(s * stripe + t * CHUNK, CHUNK)])
    plsc.subcore_barrier()

    def body(j, carry):
        pltpu.async_copy(hs_hbm.at[src_v.at[j]], buf_a, sem_a).wait()
        pltpu.sync_copy(buf_a, acc_sp.at[dst_v.at[j]], add=True)
        return carry

    lax.fori_loop(0, k, body, 0)
    plsc.subcore_barrier()
    for t in range(stripe // CHUNK):
        base = s * stripe + t * CHUNK
        pltpu.sync_copy(acc_sp.at[pl.ds(base, CHUNK)], buf_a)
        pltpu.sync_copy(buf_a, out_hbm.at[c, pl.ds(base, CHUNK)])


def _mm_body(x_ref, w_ref, degp_ref, hs_ref):
    deg = degp_ref[0, :, 0:1] + degp_ref[1, :, 0:1] + 1.0
    dis = lax.rsqrt(deg)
    h = jnp.dot(x_ref[...], w_ref[...], preferred_element_type=jnp.float32)
    hs_ref[...] = h * dis


def _fin_body(accp_ref, hs_ref, degp_ref, b_ref, out_ref):
    deg = degp_ref[0, :, 0:1] + degp_ref[1, :, 0:1] + 1.0
    dis = lax.rsqrt(deg)
    out_ref[...] = dis * (accp_ref[0] + accp_ref[1] + hs_ref[...]) + b_ref[...]


def kernel(x, edge_index, W, b):
    n, d_in = x.shape
    d = W.shape[1]
    e = edge_index.shape[1]

    n_pad = ((n // (NS * CHUNK)) + 1) * NS * CHUNK
    stripe = n_pad // NS
    trash = n

    ep = ((e + NW * CHUNK - 1) // (NW * CHUNK)) * NW * CHUNK
    k = ep // (NW * CHUNK)

    e32 = edge_index.astype(jnp.int32)
    pad = jnp.full((ep - e,), trash, dtype=jnp.int32)
    src = jnp.concatenate([e32[0], pad]).reshape(NW, k, CHUNK)
    dst = jnp.concatenate([e32[1], pad]).reshape(NW, k, CHUNK)
    x_pad = jnp.zeros((n_pad, d_in), x.dtype).at[:n].set(x)

    ones_d = jnp.ones((CHUNK, d), jnp.float32)
    zeros_d = jnp.zeros((CHUNK, d), jnp.float32)

    mesh = _sc_mesh()
    deg_part = pl.kernel(
        functools.partial(_deg_body, k, stripe),
        out_type=jax.ShapeDtypeStruct((NC, n_pad, d), jnp.float32),
        mesh=mesh,
        scratch_types=[
            pltpu.VMEM((k, CHUNK), jnp.int32),
            pltpu.VMEM((CHUNK, d), jnp.float32),
            pltpu.VMEM((CHUNK, d), jnp.float32),
            pltpu.VMEM_SHARED((n_pad, d), jnp.float32),
        ],
    )(dst, ones_d, zeros_d)

    blk = 512
    grid = (n_pad // blk,)
    hs = pl.pallas_call(
        _mm_body,
        grid=grid,
        in_specs=[
            pl.BlockSpec((blk, d_in), lambda i: (i, 0)),
            pl.BlockSpec((d_in, d), lambda i: (0, 0)),
            pl.BlockSpec((NC, blk, d), lambda i: (0, i, 0)),
        ],
        out_specs=pl.BlockSpec((blk, d), lambda i: (i, 0)),
        out_shape=jax.ShapeDtypeStruct((n_pad, d), jnp.float32),
    )(x_pad, W, deg_part)

    acc_part = pl.kernel(
        functools.partial(_edge_body, k, stripe),
        out_type=jax.ShapeDtypeStruct((NC, n_pad, d), jnp.float32),
        mesh=mesh,
        scratch_types=[
            pltpu.VMEM((k, CHUNK), jnp.int32),
            pltpu.VMEM((k, CHUNK), jnp.int32),
            pltpu.VMEM((CHUNK, d), jnp.float32),
            pltpu.VMEM_SHARED((n_pad, d), jnp.float32),
            pltpu.SemaphoreType.DMA,
        ],
    )(hs, src, dst, zeros_d)

    out_pad = pl.pallas_call(
        _fin_body,
        grid=grid,
        in_specs=[
            pl.BlockSpec((NC, blk, d), lambda i: (0, i, 0)),
            pl.BlockSpec((blk, d), lambda i: (i, 0)),
            pl.BlockSpec((NC, blk, d), lambda i: (0, i, 0)),
            pl.BlockSpec((1, d), lambda i: (0, 0)),
        ],
        out_specs=pl.BlockSpec((blk, d), lambda i: (i, 0)),
        out_shape=jax.ShapeDtypeStruct((n_pad, d), jnp.float32),
    )(acc_part, hs, deg_part, b.reshape(1, d))

    return out_pad[:n]
